# baseline (speedup 1.0000x reference)
import functools

import jax
import jax.numpy as jnp
from jax import lax
from jax.experimental import pallas as pl
from jax.experimental.pallas import tpu as pltpu
from jax.experimental.pallas import tpu_sc as plsc

_V = 1000000
_D = 64
_B = 4096
_S = 200

_CBLK = 16384
_NSTEP = pl.cdiv(_V, _CBLK)
_NW = 32
_RPW = _B // _NW


def _rowdot_body(tbl_ref, w_ref, b_ref, xv_ref, t_ref, xt_ref):
    x = tbl_ref[...]
    w8 = jnp.broadcast_to(w_ref[...], (8, _D))
    acc = jnp.dot(w8, x,
                  preferred_element_type=jnp.float32)
    t_ref[...] = ((acc[0:1] + b_ref[0]) * (1.0 / _S))[None]
    xt_ref[...] = xv_ref[...][None]


def _rowdot_shuffle(table, W, b, x):
    tt = table.T
    xv = x.T
    grid = (max(_NSTEP, _NW),)
    mat = lambda i: jnp.minimum(i, _NSTEP - 1)
    t, xt = pl.pallas_call(
        _rowdot_body,
        grid=grid,
        in_specs=[
            pl.BlockSpec((_D, _CBLK), lambda i: (0, mat(i))),
            pl.BlockSpec((1, _D), lambda i: (0, 0)),
            pl.BlockSpec(memory_space=pltpu.SMEM),
            pl.BlockSpec((_S, _RPW), lambda i: (0, jnp.minimum(i, _NW - 1))),
        ],
        out_specs=[
            pl.BlockSpec((1, 1, _CBLK), lambda i: (mat(i), 0, 0)),
            pl.BlockSpec((1, _S, _RPW),
                         lambda i: (jnp.minimum(i, _NW - 1), 0, 0)),
        ],
        out_shape=[
            jax.ShapeDtypeStruct((_NSTEP, 1, _CBLK), jnp.float32),
            jax.ShapeDtypeStruct((_NW, _S, _RPW), jnp.int32),
        ],
    )(tt, W, b, xv)
    return t.reshape(-1), xt.reshape(-1)


def _make_gather_kernel():
    info = plsc.get_sparse_core_info()
    nc, ns = info.num_cores, info.num_subcores
    nw = nc * ns
    rows_per_w = _B // nw
    idx_per_w = rows_per_w * _S
    n_grp = rows_per_w // 16

    mesh = plsc.VectorSubcoreMesh(core_axis_name="c", subcore_axis_name="s")

    tlen = _CBLK * _NSTEP
    tpw = tlen // ns

    @functools.partial(
        pl.kernel,
        out_type=jax.ShapeDtypeStruct((_B,), jnp.float32),
        mesh=mesh,
        scratch_types=[
            pltpu.VMEM((idx_per_w,), jnp.int32),
            pltpu.VMEM((idx_per_w,), jnp.float32),
            pltpu.VMEM((rows_per_w,), jnp.float32),
            pltpu.VMEM_SHARED((tlen,), jnp.float32),
            pltpu.SemaphoreType.DMA,
            pltpu.SemaphoreType.DMA,
            pltpu.SemaphoreType.DMA,
        ],
    )
    def gather_reduce(xt_hbm, t_hbm, out_hbm, idx_v, vals_v, out_v,
                      t_sh, sem, sem2, sem3):
        sid = lax.axis_index("s")
        wid = sid * nc + lax.axis_index("c")
        base = wid * idx_per_w
        idx_cp = pltpu.async_copy(xt_hbm.at[pl.ds(base, idx_per_w)],
                                  idx_v, sem2)
        pltpu.sync_copy(t_hbm.at[pl.ds(sid * tpw, tpw)],
                        t_sh.at[pl.ds(sid * tpw, tpw)])
        idx_cp.wait()
        plsc.subcore_barrier()
        half = idx_per_w // 2
        g1 = pltpu.async_copy(t_sh.at[idx_v.at[pl.ds(0, half)]],
                              vals_v.at[pl.ds(0, half)], sem)
        g2 = pltpu.async_copy(t_sh.at[idx_v.at[pl.ds(half, half)]],
                              vals_v.at[pl.ds(half, half)], sem3)

        def mkbody(goff):
            def body(c, accs):
                off = goff + c * rows_per_w
                return tuple(
                    accs[g] + vals_v[pl.ds(off + g * 16, 16)]
                    for g in range(n_grp)
                )
            return body

        g1.wait()
        accs = lax.fori_loop(
            0, _S // 2, mkbody(0),
            tuple(jnp.zeros((16,), jnp.float32) for _ in range(n_grp)))
        g2.wait()
        accs = lax.fori_loop(0, _S // 2, mkbody(half), accs)
        for g in range(n_grp):
            y = 1.0 / (1.0 + jnp.exp(-accs[g]))
            out_v[pl.ds(g * 16, 16)] = y

        pltpu.sync_copy(out_v,
                        out_hbm.at[pl.ds(wid * rows_per_w, rows_per_w)])

    return gather_reduce


def kernel(x, table, W, b):
    t, xt = _rowdot_shuffle(table, W, b, x)
    gk = _make_gather_kernel()
    out = gk(xt, t)
    return out.reshape(_B, 1)


# speedup vs baseline: 1.0862x; 1.0862x over previous
import functools

import jax
import jax.numpy as jnp
from jax import lax
from jax.experimental import pallas as pl
from jax.experimental.pallas import tpu as pltpu
from jax.experimental.pallas import tpu_sc as plsc

_V = 1000000
_D = 64
_B = 4096
_S = 200

_CBLK = 32768
_NSTEP = pl.cdiv(_V, _CBLK)
_NW = 32
_RPW = _B // _NW


def _rowdot_body(tbl_ref, w_ref, b_ref, xv_ref, t_ref, xt_ref):
    x = tbl_ref[...]
    w8 = jnp.broadcast_to(w_ref[...], (8, _D))
    acc = jnp.dot(w8, x,
                  preferred_element_type=jnp.float32)
    t_ref[...] = ((acc[0:1] + b_ref[0]) * (1.0 / _S))[None]
    xt_ref[...] = xv_ref[...][None]


def _rowdot_shuffle(table, W, b, x):
    tt = table.T
    xv = x.T
    grid = (max(_NSTEP, _NW),)
    mat = lambda i: jnp.minimum(i, _NSTEP - 1)
    t, xt = pl.pallas_call(
        _rowdot_body,
        grid=grid,
        in_specs=[
            pl.BlockSpec((_D, _CBLK), lambda i: (0, mat(i))),
            pl.BlockSpec((1, _D), lambda i: (0, 0)),
            pl.BlockSpec(memory_space=pltpu.SMEM),
            pl.BlockSpec((_S, _RPW), lambda i: (0, jnp.minimum(i, _NW - 1))),
        ],
        out_specs=[
            pl.BlockSpec((1, 1, _CBLK), lambda i: (mat(i), 0, 0)),
            pl.BlockSpec((1, _S, _RPW),
                         lambda i: (jnp.minimum(i, _NW - 1), 0, 0)),
        ],
        out_shape=[
            jax.ShapeDtypeStruct((_NSTEP, 1, _CBLK), jnp.float32),
            jax.ShapeDtypeStruct((_NW, _S, _RPW), jnp.int32),
        ],
    )(tt, W, b, xv)
    return t.reshape(-1), xt.reshape(-1)


def _make_gather_kernel():
    info = plsc.get_sparse_core_info()
    nc, ns = info.num_cores, info.num_subcores
    nw = nc * ns
    rows_per_w = _B // nw
    idx_per_w = rows_per_w * _S
    n_grp = rows_per_w // 16

    mesh = plsc.VectorSubcoreMesh(core_axis_name="c", subcore_axis_name="s")

    tlen = _CBLK * _NSTEP
    tpw = tlen // ns

    @functools.partial(
        pl.kernel,
        out_type=jax.ShapeDtypeStruct((_B,), jnp.float32),
        mesh=mesh,
        scratch_types=[
            pltpu.VMEM((idx_per_w,), jnp.int32),
            pltpu.VMEM((idx_per_w,), jnp.float32),
            pltpu.VMEM((rows_per_w,), jnp.float32),
            pltpu.VMEM_SHARED((tlen,), jnp.float32),
            pltpu.SemaphoreType.DMA,
            pltpu.SemaphoreType.DMA,
            pltpu.SemaphoreType.DMA,
        ],
    )
    def gather_reduce(xt_hbm, t_hbm, out_hbm, idx_v, vals_v, out_v,
                      t_sh, sem, sem2, sem3):
        sid = lax.axis_index("s")
        wid = sid * nc + lax.axis_index("c")
        base = wid * idx_per_w
        idx_cp = pltpu.async_copy(xt_hbm.at[pl.ds(base, idx_per_w)],
                                  idx_v, sem2)
        pltpu.sync_copy(t_hbm.at[pl.ds(sid * tpw, tpw)],
                        t_sh.at[pl.ds(sid * tpw, tpw)])
        idx_cp.wait()
        plsc.subcore_barrier()
        half = idx_per_w // 2
        g1 = pltpu.async_copy(t_sh.at[idx_v.at[pl.ds(0, half)]],
                              vals_v.at[pl.ds(0, half)], sem)
        g2 = pltpu.async_copy(t_sh.at[idx_v.at[pl.ds(half, half)]],
                              vals_v.at[pl.ds(half, half)], sem3)

        def mkbody(goff):
            def body(c, accs):
                off = goff + c * rows_per_w
                return tuple(
                    accs[g] + vals_v[pl.ds(off + g * 16, 16)]
                    for g in range(n_grp)
                )
            return body

        g1.wait()
        accs = lax.fori_loop(
            0, _S // 2, mkbody(0),
            tuple(jnp.zeros((16,), jnp.float32) for _ in range(n_grp)))
        g2.wait()
        accs = lax.fori_loop(0, _S // 2, mkbody(half), accs)
        for g in range(n_grp):
            y = 1.0 / (1.0 + jnp.exp(-accs[g]))
            out_v[pl.ds(g * 16, 16)] = y

        pltpu.sync_copy(out_v,
                        out_hbm.at[pl.ds(wid * rows_per_w, rows_per_w)])

    return gather_reduce


def kernel(x, table, W, b):
    t, xt = _rowdot_shuffle(table, W, b, x)
    gk = _make_gather_kernel()
    out = gk(xt, t)
    return out.reshape(_B, 1)


# speedup vs baseline: 1.0960x; 1.0090x over previous
import functools

import jax
import jax.numpy as jnp
from jax import lax
from jax.experimental import pallas as pl
from jax.experimental.pallas import tpu as pltpu
from jax.experimental.pallas import tpu_sc as plsc

_V = 1000000
_D = 64
_B = 4096
_S = 200

_CBLK = 31360
_NSTEP = pl.cdiv(_V, _CBLK)
_NW = 32
_RPW = _B // _NW


def _rowdot_body(tbl_ref, w_ref, b_ref, xv_ref, t_ref, xt_ref):
    x = tbl_ref[...]
    w8 = jnp.broadcast_to(w_ref[...], (8, _D))
    acc = jnp.dot(w8, x,
                  preferred_element_type=jnp.float32)
    t_ref[...] = ((acc[0:1] + b_ref[0]) * (1.0 / _S))[None]
    xt_ref[...] = xv_ref[...][None]


def _rowdot_shuffle(table, W, b, x):
    tt = table.T
    xv = x.T
    grid = (max(_NSTEP, _NW),)
    mat = lambda i: jnp.minimum(i, _NSTEP - 1)
    t, xt = pl.pallas_call(
        _rowdot_body,
        grid=grid,
        in_specs=[
            pl.BlockSpec((_D, _CBLK), lambda i: (0, mat(i))),
            pl.BlockSpec((1, _D), lambda i: (0, 0)),
            pl.BlockSpec(memory_space=pltpu.SMEM),
            pl.BlockSpec((_S, _RPW), lambda i: (0, jnp.minimum(i, _NW - 1))),
        ],
        out_specs=[
            pl.BlockSpec((1, 1, _CBLK), lambda i: (mat(i), 0, 0)),
            pl.BlockSpec((1, _S, _RPW),
                         lambda i: (jnp.minimum(i, _NW - 1), 0, 0)),
        ],
        out_shape=[
            jax.ShapeDtypeStruct((_NSTEP, 1, _CBLK), jnp.float32),
            jax.ShapeDtypeStruct((_NW, _S, _RPW), jnp.int32),
        ],
    )(tt, W, b, xv)
    return t.reshape(-1), xt.reshape(-1)


def _make_gather_kernel():
    info = plsc.get_sparse_core_info()
    nc, ns = info.num_cores, info.num_subcores
    nw = nc * ns
    rows_per_w = _B // nw
    idx_per_w = rows_per_w * _S
    n_grp = rows_per_w // 16

    mesh = plsc.VectorSubcoreMesh(core_axis_name="c", subcore_axis_name="s")

    tlen = _CBLK * _NSTEP
    tpw = tlen // ns

    @functools.partial(
        pl.kernel,
        out_type=jax.ShapeDtypeStruct((_B,), jnp.float32),
        mesh=mesh,
        scratch_types=[
            pltpu.VMEM((idx_per_w,), jnp.int32),
            pltpu.VMEM((idx_per_w,), jnp.float32),
            pltpu.VMEM((rows_per_w,), jnp.float32),
            pltpu.VMEM_SHARED((tlen,), jnp.float32),
            pltpu.SemaphoreType.DMA,
            pltpu.SemaphoreType.DMA,
            pltpu.SemaphoreType.DMA,
        ],
    )
    def gather_reduce(xt_hbm, t_hbm, out_hbm, idx_v, vals_v, out_v,
                      t_sh, sem, sem2, sem3):
        sid = lax.axis_index("s")
        wid = sid * nc + lax.axis_index("c")
        base = wid * idx_per_w
        idx_cp = pltpu.async_copy(xt_hbm.at[pl.ds(base, idx_per_w)],
                                  idx_v, sem2)
        pltpu.sync_copy(t_hbm.at[pl.ds(sid * tpw, tpw)],
                        t_sh.at[pl.ds(sid * tpw, tpw)])
        idx_cp.wait()
        plsc.subcore_barrier()
        half = idx_per_w // 2
        g1 = pltpu.async_copy(t_sh.at[idx_v.at[pl.ds(0, half)]],
                              vals_v.at[pl.ds(0, half)], sem)
        g2 = pltpu.async_copy(t_sh.at[idx_v.at[pl.ds(half, half)]],
                              vals_v.at[pl.ds(half, half)], sem3)

        def mkbody(goff):
            def body(c, accs):
                off = goff + c * rows_per_w
                return tuple(
                    accs[g] + vals_v[pl.ds(off + g * 16, 16)]
                    for g in range(n_grp)
                )
            return body

        g1.wait()
        accs = lax.fori_loop(
            0, _S // 2, mkbody(0),
            tuple(jnp.zeros((16,), jnp.float32) for _ in range(n_grp)))
        g2.wait()
        accs = lax.fori_loop(0, _S // 2, mkbody(half), accs)
        for g in range(n_grp):
            y = 1.0 / (1.0 + jnp.exp(-accs[g]))
            out_v[pl.ds(g * 16, 16)] = y

        pltpu.sync_copy(out_v,
                        out_hbm.at[pl.ds(wid * rows_per_w, rows_per_w)])

    return gather_reduce


def kernel(x, table, W, b):
    t, xt = _rowdot_shuffle(table, W, b, x)
    gk = _make_gather_kernel()
    out = gk(xt, t)
    return out.reshape(_B, 1)
